# baseline (speedup 1.0000x reference)
import jax
import jax.numpy as jnp
from jax import lax
from jax.experimental import pallas as pl
from jax.experimental.pallas import tpu as pltpu
from jax.experimental.pallas import tpu_sc as plsc

N_TILES = 8
N_FIBRES = 5000
J = 20000
B = 128
BC = 32
NCHUNK = B // BC
ROWS_PAD = N_FIBRES + 2
A_ROWS = 40064
ZSLICE = A_ROWS // 16
PER_TILE = 1280
IDXR = PER_TILE // 128
LAST = J - 15 * PER_TILE
DUMP_T = 8
DUMP_F = 22


def _body(flux2, tid, fid, etav, out2,
          A, GA, GB, GC,
          tI, fI, q0, ev,
          semZ, semS0, semS1, semS2, semO0, semO1, semO2):
    cid = lax.axis_index("c")
    sid = lax.axis_index("s")
    Slab = (GA, GB, GC)
    semS = (semS0, semS1, semS2)
    semO = (semO0, semO1, semO2)
    R = A_ROWS - 2
    Tm = A.at[pl.ds(0, R)]
    T0 = A.at[pl.ds(1, R)]
    Tp = A.at[pl.ds(2, R)]

    pltpu.sync_copy(etav, ev)
    e = ev[...]
    cc = 1.0 - 2.0 * e
    cn = e

    jbase = sid * PER_TILE
    dq = jnp.full((16,), DUMP_T * ROWS_PAD + DUMP_F, jnp.int32)

    def _round(r):
        pltpu.sync_copy(tid.at[pl.ds(jbase + r * 128, 128)], tI)
        pltpu.sync_copy(fid.at[pl.ds(jbase + r * 128, 128)], fI)
        for l in range(8):
            s = pl.ds(l * 16, 16)
            q0[r, s] = tI[s] * ROWS_PAD + fI[s]

    @pl.when(sid < 15)
    def _():
        for r in range(IDXR):
            _round(r)

    @pl.when(sid == 15)
    def _():
        for r in range(LAST // 128):
            _round(r)
        t6 = LAST // 128
        pltpu.sync_copy(tid.at[pl.ds(jbase + t6 * 128, LAST % 128)],
                        tI.at[pl.ds(0, LAST % 128)])
        pltpu.sync_copy(fid.at[pl.ds(jbase + t6 * 128, LAST % 128)],
                        fI.at[pl.ds(0, LAST % 128)])
        for l in range(8):
            s = pl.ds(l * 16, 16)
            if l * 16 < LAST % 128:
                q0[t6, s] = tI[s] * ROWS_PAD + fI[s]
            else:
                q0[t6, s] = dq
        for r in range(t6 + 1, IDXR):
            for l in range(8):
                q0[r, pl.ds(l * 16, 16)] = dq

    z16 = jnp.zeros((16,), jnp.float32)

    def _zb(i, carry):
        GC[i, pl.ds(0, 16)] = z16
        GC[i, pl.ds(16, 16)] = z16
        return carry

    def _fire_zero():
        zbase = sid * ZSLICE
        zd = [pltpu.async_copy(GC.at[pl.ds(0, 256)],
                               A.at[pl.ds(zbase + z * 256, 256)], semZ)
              for z in range(ZSLICE // 256)]
        zd.append(pltpu.async_copy(
            GC.at[pl.ds(0, ZSLICE % 256)],
            A.at[pl.ds(zbase + (ZSLICE // 256) * 256, ZSLICE % 256)], semZ))
        return zd

    def _comb(t):
        S = Slab[t]

        def body(i, carry):
            r0 = i * 4
            for u in range(4):
                r = r0 + u
                for c2 in (0, 16):
                    sl = pl.ds(c2, 16)
                    g0 = S[128 + r, sl]
                    g12 = S[r, sl] + S[256 + r, sl]
                    S[384 + r, sl] = g0 * cc + g12 * cn
            return carry

        lax.fori_loop(0, 32, body, 0)

    def _scatter(waves, co):
        ld = [pltpu.async_copy(
                  flux2.at[pl.ds(jbase + row0, nrows), pl.ds(co, BC)],
                  Slab[w].at[pl.ds(dst0, nrows)], semS[w])
              for w, (row0, nrows, dst0, _) in enumerate(waves)]
        zd = _fire_zero()
        for d in zd:
            d.wait()
        plsc.subcore_barrier()
        sc = []
        for w, (_, _, _, ss) in enumerate(waves):
            ld[w].wait()
            sc += [pltpu.async_copy(Slab[w].at[pl.ds(off, 128)],
                                    T0.at[q0.at[s]], semO[w], add=True)
                   for s, off in ss]
        for d in sc:
            d.wait()
        plsc.subcore_barrier()

    def _gather(pieces, co):
        npc = len(pieces)

        def fire_g(i):
            s = pieces[i][0]
            t = i % 3
            S = Slab[t]
            sg = semS[t]
            return [pltpu.async_copy(Tm.at[q0.at[s]], S.at[pl.ds(0, 128)], sg),
                    pltpu.async_copy(T0.at[q0.at[s]],
                                     S.at[pl.ds(128, 128)], sg),
                    pltpu.async_copy(Tp.at[q0.at[s]],
                                     S.at[pl.ds(256, 128)], sg)]

        gd = {i: fire_g(i) for i in range(min(3, npc))}
        od = {}
        for i in range(npc):
            s, rows = pieces[i]
            for d in gd[i]:
                d.wait()
            if i - 3 >= 0:
                od[i - 3].wait()
            _comb(i % 3)
            if i + 3 < npc:
                gd[i + 3] = fire_g(i + 3)
            S = Slab[i % 3]
            od[i] = pltpu.async_copy(
                S.at[pl.ds(384, rows)],
                out2.at[pl.ds(jbase + s * 128, rows), pl.ds(co, BC)],
                semO[i % 3])
        for i in range(max(0, npc - 3), npc):
            od[i].wait()

    full = [(s, 128) for s in range(IDXR)]
    short = [(s, 128) for s in range(LAST // 128)] + [(LAST // 128, LAST % 128)]
    waves_full = [(0, 512, 0, [(s, (s % 4) * 128) for s in range(0, 4)]),
                  (512, 512, 0, [(s, (s % 4) * 128) for s in range(4, 8)]),
                  (1024, 256, 256, [(8, 256), (9, 384)])]
    waves_short = [(0, 512, 0, [(s, (s % 4) * 128) for s in range(0, 4)]),
                   (512, LAST - 512, 0,
                    [(s, (s % 4) * 128) for s in range(4, 7)])]

    lax.fori_loop(0, 256, _zb, 0)

    def _chunk(k, carry):
        ci = cid * 2 + k
        co = pl.multiple_of(ci * BC, 8)

        with jax.named_scope("scat"):
            @pl.when(sid < 15)
            def _():
                _scatter(waves_full, co)

            @pl.when(sid == 15)
            def _():
                _scatter(waves_short, co)

        with jax.named_scope("gath"):
            @pl.when(sid < 15)
            def _():
                _gather(full, co)

            @pl.when(sid == 15)
            def _():
                _gather(short, co)
            plsc.subcore_barrier()

        @pl.when(k == 0)
        def _():
            lax.fori_loop(0, 256, _zb, 0)
        return carry

    lax.fori_loop(0, 2, _chunk, 0)


def _sc_call(flux2, tid, fid, etav):
    mesh = plsc.VectorSubcoreMesh(core_axis_name="c", subcore_axis_name="s")
    return pl.kernel(
        _body,
        out_type=jax.ShapeDtypeStruct((J, B), jnp.float32),
        mesh=mesh,
        compiler_params=pltpu.CompilerParams(use_tc_tiling_on_sc=False),
        scratch_types=[
            pltpu.VMEM_SHARED((A_ROWS, BC), jnp.float32),
            pltpu.VMEM((512, BC), jnp.float32),
            pltpu.VMEM((512, BC), jnp.float32),
            pltpu.VMEM((512, BC), jnp.float32),
            pltpu.VMEM((128,), jnp.int32),
            pltpu.VMEM((128,), jnp.int32),
            pltpu.VMEM((IDXR, 128), jnp.int32),
            pltpu.VMEM((16,), jnp.float32),
        ] + [pltpu.SemaphoreType.DMA] * 7,
    )(flux2, tid, fid, etav)


def kernel(flux, tile_idx, fib_idx, eta):
    etav = jnp.full((16,), eta, jnp.float32)
    out2 = _sc_call(flux.T, tile_idx, fib_idx, etav)
    return out2.T


# speedup vs baseline: 1.1823x; 1.1823x over previous
import jax
import jax.numpy as jnp
from jax import lax
from jax.experimental import pallas as pl
from jax.experimental.pallas import tpu as pltpu
from jax.experimental.pallas import tpu_sc as plsc

N_TILES = 8
N_FIBRES = 5000
J = 20000
B = 128
BC = 32
NCHUNK = B // BC
ROWS_PAD = N_FIBRES + 2
A_ROWS = 40064
ZSLICE = A_ROWS // 16
PER_TILE = 1280
IDXR = PER_TILE // 128
LAST = J - 15 * PER_TILE
DUMP_T = 8
DUMP_F = 22


def _body(flux2, q2, etav, out2,
          A, GA, GB, GC,
          q0, ev,
          semZ, semS0, semS1, semS2, semO0, semO1, semO2):
    cid = lax.axis_index("c")
    sid = lax.axis_index("s")
    Slab = (GA, GB, GC)
    semS = (semS0, semS1, semS2)
    semO = (semO0, semO1, semO2)
    R = A_ROWS - 2
    Tm = A.at[pl.ds(0, R)]
    T0 = A.at[pl.ds(1, R)]
    Tp = A.at[pl.ds(2, R)]

    pltpu.sync_copy(etav, ev)
    e = ev[...]
    cc = 1.0 - 2.0 * e
    cn = e

    jbase = sid * PER_TILE

    pltpu.sync_copy(q2.at[pl.ds(sid * IDXR, IDXR)], q0)

    z16 = jnp.zeros((16,), jnp.float32)

    def _zb(i, carry):
        GC[i, pl.ds(0, 16)] = z16
        GC[i, pl.ds(16, 16)] = z16
        return carry

    def _fire_zero():
        zbase = sid * ZSLICE
        zd = [pltpu.async_copy(GC.at[pl.ds(0, 256)],
                               A.at[pl.ds(zbase + z * 256, 256)], semZ)
              for z in range(ZSLICE // 256)]
        zd.append(pltpu.async_copy(
            GC.at[pl.ds(0, ZSLICE % 256)],
            A.at[pl.ds(zbase + (ZSLICE // 256) * 256, ZSLICE % 256)], semZ))
        return zd

    def _comb(t):
        S = Slab[t]

        def body(i, carry):
            r0 = i * 4
            for u in range(4):
                r = r0 + u
                for c2 in (0, 16):
                    sl = pl.ds(c2, 16)
                    g0 = S[128 + r, sl]
                    g12 = S[r, sl] + S[256 + r, sl]
                    S[384 + r, sl] = g0 * cc + g12 * cn
            return carry

        lax.fori_loop(0, 32, body, 0)

    def _scatter(waves, co):
        ld = []
        for w, ss in enumerate(waves):
            ld.append([pltpu.async_copy(
                           flux2.at[pl.ds(jbase + s * 128, rows),
                                    pl.ds(co, BC)],
                           Slab[w].at[pl.ds(off, rows)], semS[w])
                       for s, off, rows in ss])
        zd = _fire_zero()
        for d in zd:
            d.wait()
        plsc.subcore_barrier()
        sc = []
        for w, ss in enumerate(waves):
            for d in ld[w]:
                d.wait()
            sc += [pltpu.async_copy(Slab[w].at[pl.ds(off, 128)],
                                    T0.at[q0.at[s]], semO[w], add=True)
                   for s, off, _ in ss]
        for d in sc:
            d.wait()
        plsc.subcore_barrier()

    def _gather(pieces, co):
        npc = len(pieces)

        def fire_g(i):
            s = pieces[i][0]
            t = i % 3
            S = Slab[t]
            sg = semS[t]
            return [pltpu.async_copy(Tm.at[q0.at[s]], S.at[pl.ds(0, 128)], sg),
                    pltpu.async_copy(T0.at[q0.at[s]],
                                     S.at[pl.ds(128, 128)], sg),
                    pltpu.async_copy(Tp.at[q0.at[s]],
                                     S.at[pl.ds(256, 128)], sg)]

        gd = {i: fire_g(i) for i in range(min(3, npc))}
        od = {}
        for i in range(npc):
            s, rows = pieces[i]
            for d in gd[i]:
                d.wait()
            if i - 3 >= 0:
                od[i - 3].wait()
            _comb(i % 3)
            if i + 3 < npc:
                gd[i + 3] = fire_g(i + 3)
            S = Slab[i % 3]
            od[i] = pltpu.async_copy(
                S.at[pl.ds(384, rows)],
                out2.at[pl.ds(jbase + s * 128, rows), pl.ds(co, BC)],
                semO[i % 3])
        for i in range(max(0, npc - 3), npc):
            od[i].wait()

    full = [(s, 128) for s in range(IDXR)]
    short = [(s, 128) for s in range(LAST // 128)] + [(LAST // 128, LAST % 128)]
    waves_full = [[(s, (s % 4) * 128, 128) for s in range(0, 4)],
                  [(s, (s % 4) * 128, 128) for s in range(4, 8)],
                  [(8, 256, 128), (9, 384, 128)]]
    waves_short = [[(s, (s % 4) * 128, 128) for s in range(0, 4)],
                   [(4, 0, 128), (5, 128, 128), (6, 256, LAST % 128)]]

    lax.fori_loop(0, 256, _zb, 0)

    def _chunk(k, carry):
        ci = cid * 2 + k
        co = pl.multiple_of(ci * BC, 8)

        with jax.named_scope("scat"):
            @pl.when(sid < 15)
            def _():
                _scatter(waves_full, co)

            @pl.when(sid == 15)
            def _():
                _scatter(waves_short, co)

        with jax.named_scope("gath"):
            @pl.when(sid < 15)
            def _():
                _gather(full, co)

            @pl.when(sid == 15)
            def _():
                _gather(short, co)
            plsc.subcore_barrier()

        @pl.when(k == 0)
        def _():
            lax.fori_loop(0, 256, _zb, 0)
        return carry

    lax.fori_loop(0, 2, _chunk, 0)


def _sc_call(flux2, q2, etav):
    mesh = plsc.VectorSubcoreMesh(core_axis_name="c", subcore_axis_name="s")
    return pl.kernel(
        _body,
        out_type=jax.ShapeDtypeStruct((J, B), jnp.float32),
        mesh=mesh,
        compiler_params=pltpu.CompilerParams(use_tc_tiling_on_sc=False),
        scratch_types=[
            pltpu.VMEM_SHARED((A_ROWS, BC), jnp.float32),
            pltpu.VMEM((512, BC), jnp.float32),
            pltpu.VMEM((512, BC), jnp.float32),
            pltpu.VMEM((512, BC), jnp.float32),
            pltpu.VMEM((IDXR, 128), jnp.int32),
            pltpu.VMEM((16,), jnp.float32),
        ] + [pltpu.SemaphoreType.DMA] * 7,
    )(flux2, q2, etav)


def kernel(flux, tile_idx, fib_idx, eta):
    etav = jnp.full((16,), eta, jnp.float32)
    q = tile_idx * ROWS_PAD + fib_idx
    q2 = jnp.pad(q, (0, 16 * PER_TILE - J),
                 constant_values=DUMP_T * ROWS_PAD + DUMP_F).reshape(-1, 128)
    out2 = _sc_call(flux.T, q2, etav)
    return out2.T


# speedup vs baseline: 1.2081x; 1.0218x over previous
import jax
import jax.numpy as jnp
from jax import lax
from jax.experimental import pallas as pl
from jax.experimental.pallas import tpu as pltpu
from jax.experimental.pallas import tpu_sc as plsc

N_TILES = 8
N_FIBRES = 5000
J = 20000
B = 128
BC = 32
NCHUNK = B // BC
ROWS_PAD = N_FIBRES + 2
A_ROWS = 40064
ZSLICE = A_ROWS // 16
PER_TILE = 1280
IDXR = PER_TILE // 128
LAST = J - 15 * PER_TILE
DUMP_T = 8
DUMP_F = 22


def _body(flux2, q2, etav, out2,
          A, GA, GB, GC,
          q0, ev,
          semZ, semS0, semS1, semS2, semO0, semO1, semO2):
    cid = lax.axis_index("c")
    sid = lax.axis_index("s")
    Slab = (GA, GB, GC)
    semS = (semS0, semS1, semS2)
    semO = (semO0, semO1, semO2)
    R = A_ROWS - 2
    Tm = A.at[pl.ds(0, R)]
    T0 = A.at[pl.ds(1, R)]
    Tp = A.at[pl.ds(2, R)]

    pltpu.sync_copy(etav, ev)
    e = ev[...]
    cc = 1.0 - 2.0 * e
    cn = e

    jbase = sid * PER_TILE

    pltpu.sync_copy(q2.at[pl.ds(sid * IDXR, IDXR)], q0)

    z16 = jnp.zeros((16,), jnp.float32)

    def _zb(i, carry):
        GC[i, pl.ds(0, 16)] = z16
        GC[i, pl.ds(16, 16)] = z16
        return carry

    def _fire_zero():
        zbase = sid * ZSLICE
        zd = [pltpu.async_copy(GC.at[pl.ds(0, 256)],
                               A.at[pl.ds(zbase + z * 256, 256)], semZ)
              for z in range(ZSLICE // 256)]
        zd.append(pltpu.async_copy(
            GC.at[pl.ds(0, ZSLICE % 256)],
            A.at[pl.ds(zbase + (ZSLICE // 256) * 256, ZSLICE % 256)], semZ))
        return zd

    def _comb(t):
        S = Slab[t]

        @plsc.parallel_loop(0, 128, 1, unroll=4)
        def body(r):
            for c2 in (0, 16):
                sl = pl.ds(c2, 16)
                g0 = S[128 + r, sl]
                g12 = S[r, sl] + S[256 + r, sl]
                S[384 + r, sl] = g0 * cc + g12 * cn

    def _scatter(waves, co):
        ld = []
        for w, ss in enumerate(waves):
            ld.append([pltpu.async_copy(
                           flux2.at[pl.ds(jbase + s * 128, rows),
                                    pl.ds(co, BC)],
                           Slab[w].at[pl.ds(off, rows)], semS[w])
                       for s, off, rows in ss])
        zd = _fire_zero()
        for d in zd:
            d.wait()
        plsc.subcore_barrier()
        sc = []
        for w, ss in enumerate(waves):
            for d in ld[w]:
                d.wait()
            sc += [pltpu.async_copy(Slab[w].at[pl.ds(off, 128)],
                                    T0.at[q0.at[s]], semO[w], add=True)
                   for s, off, _ in ss]
        for d in sc:
            d.wait()
        plsc.subcore_barrier()

    def _gather(pieces, co):
        npc = len(pieces)

        def fire_g(i):
            s = pieces[i][0]
            t = i % 3
            S = Slab[t]
            sg = semS[t]
            return [pltpu.async_copy(Tm.at[q0.at[s]], S.at[pl.ds(0, 128)], sg),
                    pltpu.async_copy(T0.at[q0.at[s]],
                                     S.at[pl.ds(128, 128)], sg),
                    pltpu.async_copy(Tp.at[q0.at[s]],
                                     S.at[pl.ds(256, 128)], sg)]

        gd = {i: fire_g(i) for i in range(min(3, npc))}
        od = {}
        for i in range(npc):
            s, rows = pieces[i]
            for d in gd[i]:
                d.wait()
            if i - 3 >= 0:
                od[i - 3].wait()
            _comb(i % 3)
            if i + 3 < npc:
                gd[i + 3] = fire_g(i + 3)
            S = Slab[i % 3]
            od[i] = pltpu.async_copy(
                S.at[pl.ds(384, rows)],
                out2.at[pl.ds(jbase + s * 128, rows), pl.ds(co, BC)],
                semO[i % 3])
        for i in range(max(0, npc - 3), npc):
            od[i].wait()

    full = [(s, 128) for s in range(IDXR)]
    short = [(s, 128) for s in range(LAST // 128)] + [(LAST // 128, LAST % 128)]
    waves_full = [[(s, (s % 4) * 128, 128) for s in range(0, 4)],
                  [(s, (s % 4) * 128, 128) for s in range(4, 8)],
                  [(8, 256, 128), (9, 384, 128)]]
    waves_short = [[(s, (s % 4) * 128, 128) for s in range(0, 4)],
                   [(4, 0, 128), (5, 128, 128), (6, 256, LAST % 128)]]

    lax.fori_loop(0, 256, _zb, 0)

    def _chunk(k, carry):
        ci = cid * 2 + k
        co = pl.multiple_of(ci * BC, 8)

        @pl.when(sid < 15)
        def _():
            _scatter(waves_full, co)

        @pl.when(sid == 15)
        def _():
            _scatter(waves_short, co)

        @pl.when(sid < 15)
        def _():
            _gather(full, co)

        @pl.when(sid == 15)
        def _():
            _gather(short, co)
        plsc.subcore_barrier()

        @pl.when(k == 0)
        def _():
            lax.fori_loop(0, 256, _zb, 0)
        return carry

    lax.fori_loop(0, 2, _chunk, 0)


def _sc_call(flux2, q2, etav):
    mesh = plsc.VectorSubcoreMesh(core_axis_name="c", subcore_axis_name="s")
    return pl.kernel(
        _body,
        out_type=jax.ShapeDtypeStruct((J, B), jnp.float32),
        mesh=mesh,
        compiler_params=pltpu.CompilerParams(use_tc_tiling_on_sc=False),
        scratch_types=[
            pltpu.VMEM_SHARED((A_ROWS, BC), jnp.float32),
            pltpu.VMEM((512, BC), jnp.float32),
            pltpu.VMEM((512, BC), jnp.float32),
            pltpu.VMEM((512, BC), jnp.float32),
            pltpu.VMEM((IDXR, 128), jnp.int32),
            pltpu.VMEM((16,), jnp.float32),
        ] + [pltpu.SemaphoreType.DMA] * 7,
    )(flux2, q2, etav)


def kernel(flux, tile_idx, fib_idx, eta):
    etav = jnp.full((16,), eta, jnp.float32)
    q = tile_idx * ROWS_PAD + fib_idx
    q2 = jnp.pad(q, (0, 16 * PER_TILE - J),
                 constant_values=DUMP_T * ROWS_PAD + DUMP_F).reshape(-1, 128)
    out2 = _sc_call(flux.T, q2, etav)
    return out2.T


# speedup vs baseline: 1.2282x; 1.0166x over previous
import jax
import jax.numpy as jnp
from jax import lax
from jax.experimental import pallas as pl
from jax.experimental.pallas import tpu as pltpu
from jax.experimental.pallas import tpu_sc as plsc

N_TILES = 8
N_FIBRES = 5000
J = 20000
B = 128
BC = 32
NCHUNK = B // BC
ROWS_PAD = N_FIBRES + 2
A_ROWS = 40064
ZSLICE = A_ROWS // 16
PER_TILE = 1280
IDXR = PER_TILE // 128
LAST = J - 15 * PER_TILE
DUMP_T = 8
DUMP_F = 22


def _body(flux2, q2, etav, out2,
          A, GA, GB, GC,
          q0, ev,
          semZ, semS0, semS1, semS2, semO0, semO1, semO2,
          semA0, semA1, semA2):
    cid = lax.axis_index("c")
    sid = lax.axis_index("s")
    Slab = (GA, GB, GC)
    semS = (semS0, semS1, semS2)
    semO = (semO0, semO1, semO2)
    semA = (semA0, semA1, semA2)
    R = A_ROWS - 2
    Tm = A.at[pl.ds(0, R)]
    T0 = A.at[pl.ds(1, R)]
    Tp = A.at[pl.ds(2, R)]

    pltpu.sync_copy(etav, ev)
    e = ev[...]
    cc = 1.0 - 2.0 * e
    cn = e

    jbase = sid * PER_TILE

    pltpu.sync_copy(q2.at[pl.ds(sid * IDXR, IDXR)], q0)

    z16 = jnp.zeros((16,), jnp.float32)

    def _zb(i, carry):
        GC[i, pl.ds(0, 16)] = z16
        GC[i, pl.ds(16, 16)] = z16
        return carry

    def _fire_zero():
        zbase = sid * ZSLICE
        zd = [pltpu.async_copy(GC.at[pl.ds(0, 256)],
                               A.at[pl.ds(zbase + z * 256, 256)], semZ)
              for z in range(ZSLICE // 256)]
        zd.append(pltpu.async_copy(
            GC.at[pl.ds(0, ZSLICE % 256)],
            A.at[pl.ds(zbase + (ZSLICE // 256) * 256, ZSLICE % 256)], semZ))
        return zd

    def _comb(t):
        S = Slab[t]

        @plsc.parallel_loop(0, 128, 1, unroll=4)
        def body(r):
            for c2 in (0, 16):
                sl = pl.ds(c2, 16)
                S[384 + r, sl] = S[r, sl] * cc + S[128 + r, sl] * cn

    def _scatter(waves, co):
        ld = []
        for w, ss in enumerate(waves):
            ld.append([pltpu.async_copy(
                           flux2.at[pl.ds(jbase + s * 128, rows),
                                    pl.ds(co, BC)],
                           Slab[w].at[pl.ds(off, rows)], semS[w])
                       for s, off, rows in ss])
        zd = _fire_zero()
        for d in zd:
            d.wait()
        plsc.subcore_barrier()
        sc = []
        for w, ss in enumerate(waves):
            for d in ld[w]:
                d.wait()
            sc += [pltpu.async_copy(Slab[w].at[pl.ds(off, 128)],
                                    T0.at[q0.at[s]], semO[w], add=True)
                   for s, off, _ in ss]
        for d in sc:
            d.wait()
        plsc.subcore_barrier()

    def _gather(pieces, co):
        npc = len(pieces)

        def fire_s1(i):
            s = pieces[i][0]
            t = i % 3
            S = Slab[t]
            sg = semS[t]
            return [pltpu.async_copy(T0.at[q0.at[s]], S.at[pl.ds(0, 128)], sg),
                    pltpu.async_copy(Tm.at[q0.at[s]],
                                     S.at[pl.ds(128, 128)], sg)]

        def fire_s2(i):
            s = pieces[i][0]
            t = i % 3
            return pltpu.async_copy(Tp.at[q0.at[s]],
                                    Slab[t].at[pl.ds(128, 128)],
                                    semA[t], add=True)

        gd = {i: fire_s1(i) for i in range(min(3, npc))}
        for d in gd[0]:
            d.wait()
        g2 = {0: fire_s2(0)}
        od = {}
        for i in range(npc):
            s, rows = pieces[i]
            if i + 1 < npc:
                for d in gd[i + 1]:
                    d.wait()
                g2[i + 1] = fire_s2(i + 1)
            g2[i].wait()
            if i - 3 >= 0:
                od[i - 3].wait()
            _comb(i % 3)
            if i + 3 < npc:
                gd[i + 3] = fire_s1(i + 3)
            S = Slab[i % 3]
            od[i] = pltpu.async_copy(
                S.at[pl.ds(384, rows)],
                out2.at[pl.ds(jbase + s * 128, rows), pl.ds(co, BC)],
                semO[i % 3])
        for i in range(max(0, npc - 3), npc):
            od[i].wait()

    full = [(s, 128) for s in range(IDXR)]
    short = [(s, 128) for s in range(LAST // 128)] + [(LAST // 128, LAST % 128)]
    waves_full = [[(s, (s % 4) * 128, 128) for s in range(0, 4)],
                  [(s, (s % 4) * 128, 128) for s in range(4, 8)],
                  [(8, 256, 128), (9, 384, 128)]]
    waves_short = [[(s, (s % 4) * 128, 128) for s in range(0, 4)],
                   [(4, 0, 128), (5, 128, 128), (6, 256, LAST % 128)]]

    lax.fori_loop(0, 256, _zb, 0)

    def _chunk(k, carry):
        ci = cid * 2 + k
        co = pl.multiple_of(ci * BC, 8)

        @pl.when(sid < 15)
        def _():
            _scatter(waves_full, co)

        @pl.when(sid == 15)
        def _():
            _scatter(waves_short, co)

        @pl.when(sid < 15)
        def _():
            _gather(full, co)

        @pl.when(sid == 15)
        def _():
            _gather(short, co)
        plsc.subcore_barrier()

        @pl.when(k == 0)
        def _():
            lax.fori_loop(0, 256, _zb, 0)
        return carry

    lax.fori_loop(0, 2, _chunk, 0)


def _sc_call(flux2, q2, etav):
    mesh = plsc.VectorSubcoreMesh(core_axis_name="c", subcore_axis_name="s")
    return pl.kernel(
        _body,
        out_type=jax.ShapeDtypeStruct((J, B), jnp.float32),
        mesh=mesh,
        compiler_params=pltpu.CompilerParams(use_tc_tiling_on_sc=False),
        scratch_types=[
            pltpu.VMEM_SHARED((A_ROWS, BC), jnp.float32),
            pltpu.VMEM((512, BC), jnp.float32),
            pltpu.VMEM((512, BC), jnp.float32),
            pltpu.VMEM((512, BC), jnp.float32),
            pltpu.VMEM((IDXR, 128), jnp.int32),
            pltpu.VMEM((16,), jnp.float32),
        ] + [pltpu.SemaphoreType.DMA] * 10,
    )(flux2, q2, etav)


def kernel(flux, tile_idx, fib_idx, eta):
    etav = jnp.full((16,), eta, jnp.float32)
    q = tile_idx * ROWS_PAD + fib_idx
    q2 = jnp.pad(q, (0, 16 * PER_TILE - J),
                 constant_values=DUMP_T * ROWS_PAD + DUMP_F).reshape(-1, 128)
    out2 = _sc_call(flux.T, q2, etav)
    return out2.T
